# baseline (speedup 1.0000x reference)
import functools

import jax
import jax.numpy as jnp
from jax import lax
from jax.experimental import pallas as pl
from jax.experimental.pallas import tpu as pltpu
from jax.experimental.pallas import tpu_sc as plsc

N_NODES = 10000
N_EDGES = 320000
IN_CH = 128
HID = 32
OUT_DIM = HID // 2
CW = 16

NW = 32
CH = 128
K = 8
RPW = 80
G = RPW // K
EROWS = NW * RPW
EPAD = EROWS * CH
NPAD = 10112
RPS = NPAD // 16


def _edge_loop(with_counts, wid, srcm, dstm, table_sh, acc_sh, cnt_sh, ones_v,
               src_v, dst_v, rows_v, i_sem, g_sem, s_sem_a, s_sem_b):
    def idx_fetch(g, p):
        r0 = wid * RPW + g * K
        pltpu.async_copy(srcm.at[pl.ds(r0, K)], src_v.at[p], i_sem)
        pltpu.async_copy(dstm.at[pl.ds(r0, K)], dst_v.at[p], i_sem)

    def idx_wait(p):
        pltpu.make_async_copy(srcm.at[pl.ds(0, K)], src_v.at[p], i_sem).wait()
        pltpu.make_async_copy(dstm.at[pl.ds(0, K)], dst_v.at[p], i_sem).wait()

    def scatter_sem(p):
        return s_sem_a if p == 0 else s_sem_b

    def drain_scatters(p):
        sem = scatter_sem(p)
        for j in range(K):
            pltpu.make_async_copy(
                rows_v.at[0, j], acc_sh.at[pl.ds(0, CH)], sem).wait()
            if with_counts:
                pltpu.make_async_copy(
                    ones_v, cnt_sh.at[pl.ds(0, CH)], sem).wait()

    def run_chunk(g, p, drain_pred, prefetch_pred):
        idx_wait(p)
        gd = [
            pltpu.async_copy(
                table_sh.at[src_v.at[p, j]], rows_v.at[p, j], g_sem)
            for j in range(K)
        ]

        if drain_pred is True:
            drain_scatters(1 - p)
        else:
            @pl.when(drain_pred)
            def _():
                drain_scatters(1 - p)

        @pl.when(prefetch_pred)
        def _():
            idx_fetch(g + 1, 1 - p)

        for d in gd:
            d.wait()
        sem = scatter_sem(p)
        for j in range(K):
            pltpu.async_copy(
                rows_v.at[p, j], acc_sh.at[dst_v.at[p, j]], sem, add=True)
            if with_counts:
                pltpu.async_copy(
                    ones_v, cnt_sh.at[dst_v.at[p, j]], sem, add=True)

    idx_fetch(0, 0)

    def super_chunk_pair(h, carry):
        g0 = 2 * h
        run_chunk(g0, 0, h >= 1, g0 + 1 < G)
        run_chunk(g0 + 1, 1, True, g0 + 2 < G)
        return carry

    lax.fori_loop(0, G // 2, super_chunk_pair, 0)
    drain_scatters(1)


def _edge_pass1_body(table, srcm, dstm, z32, z16, ones_h,
                     acc_out, cnt_out, table_sh, acc_sh, cnt_sh, ones_v,
                     src_v, dst_v, rows_v, i_sem, g_sem, s_sem_a, s_sem_b):
    c = lax.axis_index("c")
    s = lax.axis_index("s")
    wid = s * 2 + c

    sl = pl.ds(s * RPS, RPS)
    pltpu.sync_copy(table.at[sl], table_sh.at[sl])
    pltpu.sync_copy(z32.at[sl], acc_sh.at[sl])
    pltpu.sync_copy(z16.at[sl], cnt_sh.at[sl])
    pltpu.sync_copy(ones_h, ones_v)
    plsc.subcore_barrier()

    _edge_loop(True, wid, srcm, dstm, table_sh, acc_sh, cnt_sh, ones_v,
               src_v, dst_v, rows_v, i_sem, g_sem, s_sem_a, s_sem_b)
    plsc.subcore_barrier()

    pltpu.sync_copy(acc_sh.at[sl], acc_out.at[c, sl])
    pltpu.sync_copy(cnt_sh.at[sl], cnt_out.at[c, sl])


_HCH = 80
_HCHUNKS = [(o, min(_HCH, RPS - o)) for o in range(0, RPS, _HCH)]


def _edge_pass2_body(acc1, cnt1, r1, srcm, dstm, z32,
                     acc_out, h1_out, table_sh, acc_sh,
                     a0_v, a1_v, r_v, c0_v, c1_v, h_v,
                     src_v, dst_v, rows_v, i_sem, g_sem, s_sem_a, s_sem_b):
    c = lax.axis_index("c")
    s = lax.axis_index("s")
    wid = s * 2 + c

    sl = pl.ds(s * RPS, RPS)
    pltpu.sync_copy(z32.at[sl], acc_sh.at[sl])

    base = s * RPS
    for off, sz in _HCHUNKS:
        row0 = base + off
        zz = pl.ds(0, sz)
        pltpu.sync_copy(acc1.at[0, pl.ds(row0, sz)], a0_v.at[zz])
        pltpu.sync_copy(acc1.at[1, pl.ds(row0, sz)], a1_v.at[zz])
        pltpu.sync_copy(r1.at[pl.ds(row0, sz)], r_v.at[zz])
        pltpu.sync_copy(cnt1.at[0, pl.ds(row0, sz)], c0_v.at[zz])
        pltpu.sync_copy(cnt1.at[1, pl.ds(row0, sz)], c1_v.at[zz])

        def rowfn(i, carry):
            cnt = c0_v[i, pl.ds(0, CW)] + c1_v[i, pl.ds(0, CW)]
            inv = 1.0 / jnp.maximum(cnt, 1.0)
            for half in (0, 16):
                hs = pl.ds(half, 16)
                v = (a0_v[i, hs] + a1_v[i, hs]) * inv + r_v[i, hs]
                h_v[i, hs] = jnp.maximum(v, 0.0)
            return carry

        lax.fori_loop(0, sz, rowfn, 0)
        pltpu.sync_copy(h_v.at[zz], table_sh.at[pl.ds(row0, sz)])

        @pl.when(c == 0)
        def _():
            pltpu.sync_copy(h_v.at[zz], h1_out.at[pl.ds(row0, sz)])

    plsc.subcore_barrier()

    _edge_loop(False, wid, srcm, dstm, table_sh, acc_sh, None, None,
               src_v, dst_v, rows_v, i_sem, g_sem, s_sem_a, s_sem_b)
    plsc.subcore_barrier()

    pltpu.sync_copy(acc_sh.at[sl], acc_out.at[c, sl])


def _edge_scratch():
    f32, i32 = jnp.float32, jnp.int32
    return [
        pltpu.VMEM((2, K, CH), i32),
        pltpu.VMEM((2, K, CH), i32),
        pltpu.VMEM((2, K, CH, HID), f32),
        pltpu.SemaphoreType.DMA,
        pltpu.SemaphoreType.DMA,
        pltpu.SemaphoreType.DMA,
        pltpu.SemaphoreType.DMA,
    ]


@functools.lru_cache(maxsize=None)
def _make_edge_pass1():
    f32 = jnp.float32
    outs = (jax.ShapeDtypeStruct((2, NPAD, HID), f32),
            jax.ShapeDtypeStruct((2, NPAD, CW), f32))
    scratch = [
        pltpu.VMEM_SHARED((NPAD, HID), f32),
        pltpu.VMEM_SHARED((NPAD, HID), f32),
        pltpu.VMEM_SHARED((NPAD, CW), f32),
        pltpu.VMEM((CH, CW), f32),
    ] + _edge_scratch()
    mesh = plsc.VectorSubcoreMesh(core_axis_name="c", subcore_axis_name="s")
    return pl.kernel(
        _edge_pass1_body,
        out_type=outs,
        mesh=mesh,
        scratch_types=scratch,
        compiler_params=pltpu.CompilerParams(use_tc_tiling_on_sc=False),
        name="sage_edge_pass1",
    )


@functools.lru_cache(maxsize=None)
def _make_edge_pass2():
    f32 = jnp.float32
    outs = (jax.ShapeDtypeStruct((2, NPAD, HID), f32),
            jax.ShapeDtypeStruct((NPAD, HID), f32))
    scratch = [
        pltpu.VMEM_SHARED((NPAD, HID), f32),
        pltpu.VMEM_SHARED((NPAD, HID), f32),
        pltpu.VMEM((_HCH, HID), f32),
        pltpu.VMEM((_HCH, HID), f32),
        pltpu.VMEM((_HCH, HID), f32),
        pltpu.VMEM((_HCH, CW), f32),
        pltpu.VMEM((_HCH, CW), f32),
        pltpu.VMEM((_HCH, HID), f32),
    ] + _edge_scratch()
    mesh = plsc.VectorSubcoreMesh(core_axis_name="c", subcore_axis_name="s")
    return pl.kernel(
        _edge_pass2_body,
        out_type=outs,
        mesh=mesh,
        scratch_types=scratch,
        compiler_params=pltpu.CompilerParams(use_tc_tiling_on_sc=False),
        name="sage_edge_pass2",
    )


def _dot_t(a, w):
    return lax.dot_general(a, w, (((1,), (1,)), ((), ())),
                           preferred_element_type=jnp.float32)


def _pad_rows(v):
    return jnp.concatenate(
        [v, jnp.zeros((NPAD - N_NODES, v.shape[1]), v.dtype)], axis=0)


def _pre_body(x, wl, wr, bl, p_out, r_out):
    xv = x[...]
    p_out[...] = _pad_rows(_dot_t(xv, wl[...]))
    r_out[...] = _pad_rows(_dot_t(xv, wr[...]) + bl[...])


def _post_body(accA, accB, cntA, cntB, h1, wl2, bl2, wr2, wh, bh, z_out):
    agg = accA[...][:N_NODES] + accB[...][:N_NODES]
    cnt = cntA[...][:N_NODES, 0:1] + cntB[...][:N_NODES, 0:1]
    mean2 = agg / jnp.maximum(cnt, 1.0)
    h2 = jnp.maximum(
        _dot_t(mean2, wl2[...]) + _dot_t(h1[...][:N_NODES], wr2[...])
        + bl2[...], 0.0)
    z_out[...] = _dot_t(h2, wh[...]) + bh[...]


_f32 = jnp.float32

_pre = pl.pallas_call(
    _pre_body,
    out_shape=(jax.ShapeDtypeStruct((NPAD, HID), _f32),
               jax.ShapeDtypeStruct((NPAD, HID), _f32)),
)

_post = pl.pallas_call(
    _post_body,
    out_shape=jax.ShapeDtypeStruct((N_NODES, OUT_DIM), _f32),
)


def kernel(x, edge_index, Wl1, bl1, Wr1, Wl2, bl2, Wr2, Wh, bh):
    src = edge_index[0].astype(jnp.int32)
    dst = edge_index[1].astype(jnp.int32)
    npad = EPAD - N_EDGES
    srcm = jnp.concatenate(
        [src, jnp.zeros((npad,), jnp.int32)]).reshape(EROWS, CH)
    dstm = jnp.concatenate(
        [dst, jnp.full((npad,), NPAD - 8, jnp.int32)]).reshape(EROWS, CH)
    z32 = jnp.zeros((NPAD, HID), _f32)
    z16 = jnp.zeros((NPAD, CW), _f32)
    ones_h = jnp.ones((CH, CW), _f32)

    p1, r1 = _pre(x, Wl1, Wr1, bl1.reshape(1, HID))
    acc1, cnt1 = _make_edge_pass1()(p1, srcm, dstm, z32, z16, ones_h)
    acc2, h1 = _make_edge_pass2()(acc1, cnt1, r1, srcm, dstm, z32)
    z = _post(acc2[0], acc2[1], cnt1[0], cnt1[1], h1,
              Wl2, bl2.reshape(1, HID), Wr2, Wh, bh.reshape(1, OUT_DIM))
    return z


# speedup vs baseline: 1.0980x; 1.0980x over previous
import functools

import jax
import jax.numpy as jnp
from jax import lax
from jax.experimental import pallas as pl
from jax.experimental.pallas import tpu as pltpu
from jax.experimental.pallas import tpu_sc as plsc

N_NODES = 10000
N_EDGES = 320000
IN_CH = 128
HID = 32
OUT_DIM = HID // 2
CW = 16

NW = 32
CH = 128
K = 4
RPW = 80
G = RPW // K
EROWS = NW * RPW
EPAD = EROWS * CH
NPAD = 10112
RPS = NPAD // 16


def _edge_loop(with_counts, wid, srcm, dstm, table_sh, acc_sh, cnt_sh, ones_v,
               src_v, dst_v, rows_v, i_sem, g_sem, s_sem_a, s_sem_b):
    def idx_fetch(g, p):
        r0 = wid * RPW + g * K
        pltpu.async_copy(srcm.at[pl.ds(r0, K)], src_v.at[p], i_sem)
        pltpu.async_copy(dstm.at[pl.ds(r0, K)], dst_v.at[p], i_sem)

    def idx_wait(p):
        pltpu.make_async_copy(srcm.at[pl.ds(0, K)], src_v.at[p], i_sem).wait()
        pltpu.make_async_copy(dstm.at[pl.ds(0, K)], dst_v.at[p], i_sem).wait()

    def scatter_sem(p):
        return s_sem_a if p == 0 else s_sem_b

    def drain_scatters(p):
        sem = scatter_sem(p)
        for j in range(K):
            pltpu.make_async_copy(
                rows_v.at[0, j], acc_sh.at[pl.ds(0, CH)], sem).wait()
            if with_counts:
                pltpu.make_async_copy(
                    ones_v, cnt_sh.at[pl.ds(0, CH)], sem).wait()

    def run_chunk(g, p, drain_pred, prefetch_pred):
        idx_wait(p)
        gd = [
            pltpu.async_copy(
                table_sh.at[src_v.at[p, j]], rows_v.at[p, j], g_sem)
            for j in range(K)
        ]

        if drain_pred is True:
            drain_scatters(1 - p)
        else:
            @pl.when(drain_pred)
            def _():
                drain_scatters(1 - p)

        @pl.when(prefetch_pred)
        def _():
            idx_fetch(g + 1, 1 - p)

        for d in gd:
            d.wait()
        sem = scatter_sem(p)
        for j in range(K):
            pltpu.async_copy(
                rows_v.at[p, j], acc_sh.at[dst_v.at[p, j]], sem, add=True)
            if with_counts:
                pltpu.async_copy(
                    ones_v, cnt_sh.at[dst_v.at[p, j]], sem, add=True)

    idx_fetch(0, 0)

    def super_chunk_pair(h, carry):
        g0 = 2 * h
        run_chunk(g0, 0, h >= 1, g0 + 1 < G)
        run_chunk(g0 + 1, 1, True, g0 + 2 < G)
        return carry

    lax.fori_loop(0, G // 2, super_chunk_pair, 0)
    drain_scatters(1)


def _edge_pass1_body(table, srcm, dstm, z32, z16, ones_h,
                     acc_out, cnt_out, table_sh, acc_sh, cnt_sh, ones_v,
                     src_v, dst_v, rows_v, i_sem, g_sem, s_sem_a, s_sem_b):
    c = lax.axis_index("c")
    s = lax.axis_index("s")
    wid = s * 2 + c

    sl = pl.ds(s * RPS, RPS)
    pltpu.sync_copy(table.at[sl], table_sh.at[sl])
    pltpu.sync_copy(z32.at[sl], acc_sh.at[sl])
    pltpu.sync_copy(z16.at[sl], cnt_sh.at[sl])
    pltpu.sync_copy(ones_h, ones_v)
    plsc.subcore_barrier()

    _edge_loop(True, wid, srcm, dstm, table_sh, acc_sh, cnt_sh, ones_v,
               src_v, dst_v, rows_v, i_sem, g_sem, s_sem_a, s_sem_b)
    plsc.subcore_barrier()

    pltpu.sync_copy(acc_sh.at[sl], acc_out.at[c, sl])
    pltpu.sync_copy(cnt_sh.at[sl], cnt_out.at[c, sl])


_HCH = 256
_HCHUNKS = [(0, 256), (256, 256), (512, 120)]


def _edge_pass2_body(acc1, cnt1, r1, srcm, dstm, z32,
                     acc_out, h1_out, table_sh, acc_sh,
                     a0_v, a1_v, r_v, c0_v, c1_v, h_v,
                     src_v, dst_v, rows_v, i_sem, g_sem, s_sem_a, s_sem_b):
    c = lax.axis_index("c")
    s = lax.axis_index("s")
    wid = s * 2 + c

    sl = pl.ds(s * RPS, RPS)
    pltpu.sync_copy(z32.at[sl], acc_sh.at[sl])

    base = s * RPS
    for ci, (off, sz) in enumerate(_HCHUNKS):
        row0 = base + off
        zz = pl.ds(0, sz)
        loads = [
            pltpu.async_copy(acc1.at[0, pl.ds(row0, sz)], a0_v.at[zz], i_sem),
            pltpu.async_copy(acc1.at[1, pl.ds(row0, sz)], a1_v.at[zz], i_sem),
            pltpu.async_copy(r1.at[pl.ds(row0, sz)], r_v.at[zz], i_sem),
            pltpu.async_copy(cnt1.at[0, pl.ds(row0, sz)], c0_v.at[zz], i_sem),
            pltpu.async_copy(cnt1.at[1, pl.ds(row0, sz)], c1_v.at[zz], i_sem),
        ]
        for d in loads:
            d.wait()

        def rowfn(i, carry):
            cnt = c0_v[i, pl.ds(0, CW)] + c1_v[i, pl.ds(0, CW)]
            inv = 1.0 / jnp.maximum(cnt, 1.0)
            for half in (0, 16):
                hs = pl.ds(half, 16)
                v = (a0_v[i, hs] + a1_v[i, hs]) * inv + r_v[i, hs]
                h_v[i, hs] = jnp.maximum(v, 0.0)
            return carry

        lax.fori_loop(0, sz, rowfn, 0)
        pltpu.sync_copy(h_v.at[zz], table_sh.at[pl.ds(row0, sz)])

        @pl.when(c == 0)
        def _():
            pltpu.sync_copy(h_v.at[zz], h1_out.at[pl.ds(row0, sz)])

    plsc.subcore_barrier()

    _edge_loop(False, wid, srcm, dstm, table_sh, acc_sh, None, None,
               src_v, dst_v, rows_v, i_sem, g_sem, s_sem_a, s_sem_b)
    plsc.subcore_barrier()

    pltpu.sync_copy(acc_sh.at[sl], acc_out.at[c, sl])


def _edge_scratch():
    f32, i32 = jnp.float32, jnp.int32
    return [
        pltpu.VMEM((2, K, CH), i32),
        pltpu.VMEM((2, K, CH), i32),
        pltpu.VMEM((2, K, CH, HID), f32),
        pltpu.SemaphoreType.DMA,
        pltpu.SemaphoreType.DMA,
        pltpu.SemaphoreType.DMA,
        pltpu.SemaphoreType.DMA,
    ]


@functools.lru_cache(maxsize=None)
def _make_edge_pass1():
    f32 = jnp.float32
    outs = (jax.ShapeDtypeStruct((2, NPAD, HID), f32),
            jax.ShapeDtypeStruct((2, NPAD, CW), f32))
    scratch = [
        pltpu.VMEM_SHARED((NPAD, HID), f32),
        pltpu.VMEM_SHARED((NPAD, HID), f32),
        pltpu.VMEM_SHARED((NPAD, CW), f32),
        pltpu.VMEM((CH, CW), f32),
    ] + _edge_scratch()
    mesh = plsc.VectorSubcoreMesh(core_axis_name="c", subcore_axis_name="s")
    return pl.kernel(
        _edge_pass1_body,
        out_type=outs,
        mesh=mesh,
        scratch_types=scratch,
        compiler_params=pltpu.CompilerParams(use_tc_tiling_on_sc=False),
        name="sage_edge_pass1",
    )


@functools.lru_cache(maxsize=None)
def _make_edge_pass2():
    f32 = jnp.float32
    outs = (jax.ShapeDtypeStruct((2, NPAD, HID), f32),
            jax.ShapeDtypeStruct((NPAD, HID), f32))
    scratch = [
        pltpu.VMEM_SHARED((NPAD, HID), f32),
        pltpu.VMEM_SHARED((NPAD, HID), f32),
        pltpu.VMEM((_HCH, HID), f32),
        pltpu.VMEM((_HCH, HID), f32),
        pltpu.VMEM((_HCH, HID), f32),
        pltpu.VMEM((_HCH, CW), f32),
        pltpu.VMEM((_HCH, CW), f32),
        pltpu.VMEM((_HCH, HID), f32),
    ] + _edge_scratch()
    mesh = plsc.VectorSubcoreMesh(core_axis_name="c", subcore_axis_name="s")
    return pl.kernel(
        _edge_pass2_body,
        out_type=outs,
        mesh=mesh,
        scratch_types=scratch,
        compiler_params=pltpu.CompilerParams(use_tc_tiling_on_sc=False),
        name="sage_edge_pass2",
    )


def _dot_t(a, w):
    return lax.dot_general(a, w, (((1,), (1,)), ((), ())),
                           preferred_element_type=jnp.float32)


def _pad_rows(v):
    return jnp.concatenate(
        [v, jnp.zeros((NPAD - N_NODES, v.shape[1]), v.dtype)], axis=0)


def _pre_body(x, wl, wr, bl, p_out, r_out):
    xv = x[...]
    p_out[...] = _pad_rows(_dot_t(xv, wl[...]))
    r_out[...] = _pad_rows(_dot_t(xv, wr[...]) + bl[...])


def _post_body(accA, accB, cntA, cntB, h1, wl2, bl2, wr2, wh, bh, z_out):
    agg = accA[...][:N_NODES] + accB[...][:N_NODES]
    cnt = cntA[...][:N_NODES, 0:1] + cntB[...][:N_NODES, 0:1]
    mean2 = agg / jnp.maximum(cnt, 1.0)
    h2 = jnp.maximum(
        _dot_t(mean2, wl2[...]) + _dot_t(h1[...][:N_NODES], wr2[...])
        + bl2[...], 0.0)
    z_out[...] = _dot_t(h2, wh[...]) + bh[...]


_f32 = jnp.float32

_pre = pl.pallas_call(
    _pre_body,
    out_shape=(jax.ShapeDtypeStruct((NPAD, HID), _f32),
               jax.ShapeDtypeStruct((NPAD, HID), _f32)),
)

_post = pl.pallas_call(
    _post_body,
    out_shape=jax.ShapeDtypeStruct((N_NODES, OUT_DIM), _f32),
)


def kernel(x, edge_index, Wl1, bl1, Wr1, Wl2, bl2, Wr2, Wh, bh):
    src = edge_index[0].astype(jnp.int32)
    dst = edge_index[1].astype(jnp.int32)
    npad = EPAD - N_EDGES
    srcm = jnp.concatenate(
        [src, jnp.zeros((npad,), jnp.int32)]).reshape(EROWS, CH)
    dstm = jnp.concatenate(
        [dst, jnp.full((npad,), NPAD - 8, jnp.int32)]).reshape(EROWS, CH)
    z32 = jnp.zeros((NPAD, HID), _f32)
    z16 = jnp.zeros((NPAD, CW), _f32)
    ones_h = jnp.ones((CH, CW), _f32)

    p1, r1 = _pre(x, Wl1, Wr1, bl1.reshape(1, HID))
    acc1, cnt1 = _make_edge_pass1()(p1, srcm, dstm, z32, z16, ones_h)
    acc2, h1 = _make_edge_pass2()(acc1, cnt1, r1, srcm, dstm, z32)
    z = _post(acc2[0], acc2[1], cnt1[0], cnt1[1], h1,
              Wl2, bl2.reshape(1, HID), Wr2, Wh, bh.reshape(1, OUT_DIM))
    return z


# speedup vs baseline: 1.1306x; 1.0297x over previous
import functools

import jax
import jax.numpy as jnp
from jax import lax
from jax.experimental import pallas as pl
from jax.experimental.pallas import tpu as pltpu
from jax.experimental.pallas import tpu_sc as plsc

N_NODES = 10000
N_EDGES = 320000
IN_CH = 128
HID = 32
OUT_DIM = HID // 2
CW = 16

NW = 32
CH = 128
K1 = 8
K2 = 4
RPW = 80
EROWS = NW * RPW
EPAD = EROWS * CH
NPAD = 10112
RPS = NPAD // 16


def _idx_fetch(kk, wid, srcm, dstm, src_v, dst_v, i_sem, g, p):
    r0 = wid * RPW + g * kk
    pltpu.async_copy(srcm.at[pl.ds(r0, kk)], src_v.at[p], i_sem)
    pltpu.async_copy(dstm.at[pl.ds(r0, kk)], dst_v.at[p], i_sem)


def _edge_loop(with_counts, kk, wid, srcm, dstm, table_sh, acc_sh, cnt_sh,
               ones_v, src_v, dst_v, rows_v, i_sem, g_sem, s_sem_a, s_sem_b):
    K = kk
    G = RPW // kk

    def idx_fetch(g, p):
        _idx_fetch(kk, wid, srcm, dstm, src_v, dst_v, i_sem, g, p)

    def idx_wait(p):
        pltpu.make_async_copy(srcm.at[pl.ds(0, K)], src_v.at[p], i_sem).wait()
        pltpu.make_async_copy(dstm.at[pl.ds(0, K)], dst_v.at[p], i_sem).wait()

    def scatter_sem(p):
        return s_sem_a if p == 0 else s_sem_b

    def drain_scatters(p):
        sem = scatter_sem(p)
        for j in range(K):
            pltpu.make_async_copy(
                rows_v.at[0, j], acc_sh.at[pl.ds(0, CH)], sem).wait()
            if with_counts:
                pltpu.make_async_copy(
                    ones_v, cnt_sh.at[pl.ds(0, CH)], sem).wait()

    def run_chunk(g, p, drain_pred, prefetch_pred):
        idx_wait(p)
        gd = [
            pltpu.async_copy(
                table_sh.at[src_v.at[p, j]], rows_v.at[p, j], g_sem)
            for j in range(K)
        ]

        if drain_pred is True:
            drain_scatters(1 - p)
        else:
            @pl.when(drain_pred)
            def _():
                drain_scatters(1 - p)

        @pl.when(prefetch_pred)
        def _():
            idx_fetch(g + 1, 1 - p)

        for d in gd:
            d.wait()
        sem = scatter_sem(p)
        for j in range(K):
            pltpu.async_copy(
                rows_v.at[p, j], acc_sh.at[dst_v.at[p, j]], sem, add=True)
            if with_counts:
                pltpu.async_copy(
                    ones_v, cnt_sh.at[dst_v.at[p, j]], sem, add=True)

    def super_chunk_pair(h, carry):
        g0 = 2 * h
        run_chunk(g0, 0, h >= 1, g0 + 1 < G)
        run_chunk(g0 + 1, 1, True, g0 + 2 < G)
        return carry

    lax.fori_loop(0, G // 2, super_chunk_pair, 0)
    drain_scatters(1)


def _edge_pass1_body(table, srcm, dstm, z32, z16, ones_h,
                     acc_out, cnt_out, table_sh, acc_sh, cnt_sh, ones_v,
                     src_v, dst_v, rows_v, i_sem, g_sem, s_sem_a, s_sem_b):
    c = lax.axis_index("c")
    s = lax.axis_index("s")
    wid = s * 2 + c

    _idx_fetch(K1, wid, srcm, dstm, src_v, dst_v, i_sem, 0, 0)

    sl = pl.ds(s * RPS, RPS)
    stage = [
        pltpu.async_copy(table.at[sl], table_sh.at[sl], g_sem),
        pltpu.async_copy(z32.at[sl], acc_sh.at[sl], g_sem),
        pltpu.async_copy(z16.at[sl], cnt_sh.at[sl], g_sem),
        pltpu.async_copy(ones_h, ones_v, g_sem),
    ]
    for d in stage:
        d.wait()
    plsc.subcore_barrier()

    _edge_loop(True, K1, wid, srcm, dstm, table_sh, acc_sh, cnt_sh, ones_v,
               src_v, dst_v, rows_v, i_sem, g_sem, s_sem_a, s_sem_b)
    plsc.subcore_barrier()

    pltpu.sync_copy(acc_sh.at[sl], acc_out.at[c, sl])
    pltpu.sync_copy(cnt_sh.at[sl], cnt_out.at[c, sl])


_HCH = 256
_HCHUNKS = [(0, 256), (256, 256), (512, 120)]


def _edge_pass2_body(acc1, cnt1, r1, srcm, dstm, z32,
                     acc_out, h1_out, table_sh, acc_sh,
                     a0_v, a1_v, r_v, c0_v, c1_v, h_v,
                     src_v, dst_v, rows_v, i_sem, g_sem, s_sem_a, s_sem_b):
    c = lax.axis_index("c")
    s = lax.axis_index("s")
    wid = s * 2 + c

    _idx_fetch(K2, wid, srcm, dstm, src_v, dst_v, i_sem, 0, 0)

    sl = pl.ds(s * RPS, RPS)
    zd = pltpu.async_copy(z32.at[sl], acc_sh.at[sl], g_sem)

    base = s * RPS
    for ci, (off, sz) in enumerate(_HCHUNKS):
        row0 = base + off
        zz = pl.ds(0, sz)
        loads = [
            pltpu.async_copy(acc1.at[0, pl.ds(row0, sz)], a0_v.at[zz], s_sem_a),
            pltpu.async_copy(acc1.at[1, pl.ds(row0, sz)], a1_v.at[zz], s_sem_a),
            pltpu.async_copy(r1.at[pl.ds(row0, sz)], r_v.at[zz], s_sem_a),
            pltpu.async_copy(cnt1.at[0, pl.ds(row0, sz)], c0_v.at[zz], s_sem_a),
            pltpu.async_copy(cnt1.at[1, pl.ds(row0, sz)], c1_v.at[zz], s_sem_a),
        ]
        for d in loads:
            d.wait()

        def rowfn(i, carry):
            cnt = c0_v[i, pl.ds(0, CW)] + c1_v[i, pl.ds(0, CW)]
            inv = 1.0 / jnp.maximum(cnt, 1.0)
            for half in (0, 16):
                hs = pl.ds(half, 16)
                v = (a0_v[i, hs] + a1_v[i, hs]) * inv + r_v[i, hs]
                h_v[i, hs] = jnp.maximum(v, 0.0)
            return carry

        lax.fori_loop(0, sz, rowfn, 0)
        pltpu.sync_copy(h_v.at[zz], table_sh.at[pl.ds(row0, sz)])

        @pl.when(c == 0)
        def _():
            pltpu.sync_copy(h_v.at[zz], h1_out.at[pl.ds(row0, sz)])

    zd.wait()
    plsc.subcore_barrier()

    _edge_loop(False, K2, wid, srcm, dstm, table_sh, acc_sh, None, None,
               src_v, dst_v, rows_v, i_sem, g_sem, s_sem_a, s_sem_b)
    plsc.subcore_barrier()

    pltpu.sync_copy(acc_sh.at[sl], acc_out.at[c, sl])


def _edge_scratch(kk):
    f32, i32 = jnp.float32, jnp.int32
    return [
        pltpu.VMEM((2, kk, CH), i32),
        pltpu.VMEM((2, kk, CH), i32),
        pltpu.VMEM((2, kk, CH, HID), f32),
        pltpu.SemaphoreType.DMA,
        pltpu.SemaphoreType.DMA,
        pltpu.SemaphoreType.DMA,
        pltpu.SemaphoreType.DMA,
    ]


@functools.lru_cache(maxsize=None)
def _make_edge_pass1():
    f32 = jnp.float32
    outs = (jax.ShapeDtypeStruct((2, NPAD, HID), f32),
            jax.ShapeDtypeStruct((2, NPAD, CW), f32))
    scratch = [
        pltpu.VMEM_SHARED((NPAD, HID), f32),
        pltpu.VMEM_SHARED((NPAD, HID), f32),
        pltpu.VMEM_SHARED((NPAD, CW), f32),
        pltpu.VMEM((CH, CW), f32),
    ] + _edge_scratch(K1)
    mesh = plsc.VectorSubcoreMesh(core_axis_name="c", subcore_axis_name="s")
    return pl.kernel(
        _edge_pass1_body,
        out_type=outs,
        mesh=mesh,
        scratch_types=scratch,
        compiler_params=pltpu.CompilerParams(use_tc_tiling_on_sc=False),
        name="sage_edge_pass1",
    )


@functools.lru_cache(maxsize=None)
def _make_edge_pass2():
    f32 = jnp.float32
    outs = (jax.ShapeDtypeStruct((2, NPAD, HID), f32),
            jax.ShapeDtypeStruct((NPAD, HID), f32))
    scratch = [
        pltpu.VMEM_SHARED((NPAD, HID), f32),
        pltpu.VMEM_SHARED((NPAD, HID), f32),
        pltpu.VMEM((_HCH, HID), f32),
        pltpu.VMEM((_HCH, HID), f32),
        pltpu.VMEM((_HCH, HID), f32),
        pltpu.VMEM((_HCH, CW), f32),
        pltpu.VMEM((_HCH, CW), f32),
        pltpu.VMEM((_HCH, HID), f32),
    ] + _edge_scratch(K2)
    mesh = plsc.VectorSubcoreMesh(core_axis_name="c", subcore_axis_name="s")
    return pl.kernel(
        _edge_pass2_body,
        out_type=outs,
        mesh=mesh,
        scratch_types=scratch,
        compiler_params=pltpu.CompilerParams(use_tc_tiling_on_sc=False),
        name="sage_edge_pass2",
    )


def _dot_t(a, w):
    return lax.dot_general(a, w, (((1,), (1,)), ((), ())),
                           preferred_element_type=jnp.float32)


def _pad_rows(v):
    return jnp.concatenate(
        [v, jnp.zeros((NPAD - N_NODES, v.shape[1]), v.dtype)], axis=0)


def _pre_body(x, wl, wr, bl, p_out, r_out):
    xv = x[...]
    p_out[...] = _pad_rows(_dot_t(xv, wl[...]))
    r_out[...] = _pad_rows(_dot_t(xv, wr[...]) + bl[...])


def _post_body(accA, accB, cntA, cntB, h1, wl2, bl2, wr2, wh, bh, z_out):
    agg = accA[...][:N_NODES] + accB[...][:N_NODES]
    cnt = cntA[...][:N_NODES, 0:1] + cntB[...][:N_NODES, 0:1]
    mean2 = agg / jnp.maximum(cnt, 1.0)
    h2 = jnp.maximum(
        _dot_t(mean2, wl2[...]) + _dot_t(h1[...][:N_NODES], wr2[...])
        + bl2[...], 0.0)
    z_out[...] = _dot_t(h2, wh[...]) + bh[...]


_f32 = jnp.float32

_pre = pl.pallas_call(
    _pre_body,
    out_shape=(jax.ShapeDtypeStruct((NPAD, HID), _f32),
               jax.ShapeDtypeStruct((NPAD, HID), _f32)),
)

_post = pl.pallas_call(
    _post_body,
    out_shape=jax.ShapeDtypeStruct((N_NODES, OUT_DIM), _f32),
)


def kernel(x, edge_index, Wl1, bl1, Wr1, Wl2, bl2, Wr2, Wh, bh):
    src = edge_index[0].astype(jnp.int32)
    dst = edge_index[1].astype(jnp.int32)
    npad = EPAD - N_EDGES
    srcm = jnp.concatenate(
        [src, jnp.zeros((npad,), jnp.int32)]).reshape(EROWS, CH)
    dstm = jnp.concatenate(
        [dst, jnp.full((npad,), NPAD - 8, jnp.int32)]).reshape(EROWS, CH)
    z32 = jnp.zeros((NPAD, HID), _f32)
    z16 = jnp.zeros((NPAD, CW), _f32)
    ones_h = jnp.ones((CH, CW), _f32)

    p1, r1 = _pre(x, Wl1, Wr1, bl1.reshape(1, HID))
    acc1, cnt1 = _make_edge_pass1()(p1, srcm, dstm, z32, z16, ones_h)
    acc2, h1 = _make_edge_pass2()(acc1, cnt1, r1, srcm, dstm, z32)
    z = _post(acc2[0], acc2[1], cnt1[0], cnt1[1], h1,
              Wl2, bl2.reshape(1, HID), Wr2, Wh, bh.reshape(1, OUT_DIM))
    return z
